# Initial kernel scaffold; baseline (speedup 1.0000x reference)
import jax
import jax.numpy as jnp
from jax.experimental import pallas as pl


def kernel(x, edge_attr, W1, b1, g1, be1, W2, b2, g2, be2, fc1_W, fc1_b, fc2_W, fc2_b, edge_index, batch):
    raise NotImplementedError("write your pallas kernel here")


# speedup vs baseline: 9.2196x; 9.2196x over previous
import functools

import numpy as np
import jax
import jax.numpy as jnp
from jax import lax
from jax.experimental import pallas as pl
from jax.experimental.pallas import tpu as pltpu
from jax.experimental.pallas import tpu_sc as plsc

_N = 10000
_E = 320000
_D = 128
_EAP = 32
_G = 250
_NPG = 40
_MLP = 256
_NC = 10
_BN_S = float(1.0 / np.sqrt(1.0 + 1e-5))

_NW = 32
_PER_W = _E // _NW
_C = 80
_STEPS = _PER_W // _C
_PAIRS = (_STEPS - 1) // 2
_RPT = 624
_TAIL = _N - 16 * _RPT
_TAIL_OFF = 16 * _RPT

_mesh = plsc.VectorSubcoreMesh(core_axis_name="c", subcore_axis_name="s")


def _make_sc_agg(with_ea):
    outh_t = jax.ShapeDtypeStruct((2, _N, _D), jnp.float32)
    out_type = [outh_t] if with_ea else outh_t
    scratch = [
        pltpu.VMEM((2, _C), jnp.int32),
        pltpu.VMEM((2, _C), jnp.int32),
        pltpu.VMEM((_C, _D), jnp.float32),
        pltpu.VMEM((_C, _D), jnp.float32),
        pltpu.VMEM_SHARED((_N, _D), jnp.float32),
        pltpu.SemaphoreType.DMA,
        pltpu.SemaphoreType.DMA,
        pltpu.SemaphoreType.DMA,
        pltpu.SemaphoreType.DMA,
    ]
    if with_ea:
        out_type.append(jax.ShapeDtypeStruct((2, _N, _EAP), jnp.float32))
        scratch += [
            pltpu.VMEM((_C, _EAP), jnp.float32),
            pltpu.VMEM((_C, _EAP), jnp.float32),
            pltpu.VMEM_SHARED((_N, _EAP), jnp.float32),
            pltpu.SemaphoreType.DMA,
            pltpu.SemaphoreType.DMA,
            pltpu.SemaphoreType.DMA,
        ]

    @functools.partial(
        pl.kernel,
        mesh=_mesh,
        compiler_params=pltpu.CompilerParams(use_tc_tiling_on_sc=False),
        out_type=out_type,
        scratch_types=scratch,
    )
    def k(*refs):
        if with_ea:
            (h_hbm, idx_hbm, ea_hbm, z128_hbm, z32_hbm,
             outh_hbm, outea_hbm,
             idx0, idx1, rows0, rows1, acch, isem, gsem, ss0, ss1,
             ea0, ea1, accea, easem, es0, es1) = refs
        else:
            (h_hbm, idx_hbm, z128_hbm,
             outh_hbm,
             idx0, idx1, rows0, rows1, acch, isem, gsem, ss0, ss1) = refs
            ea0 = ea1 = accea = easem = es0 = es1 = None

        c = lax.axis_index("c")
        s = lax.axis_index("s")
        wid = s * 2 + c
        rbase = s * _RPT

        pltpu.sync_copy(z128_hbm.at[pl.ds(rbase, _RPT)], acch.at[pl.ds(rbase, _RPT)])
        if with_ea:
            pltpu.sync_copy(z32_hbm.at[pl.ds(rbase, _RPT)], accea.at[pl.ds(rbase, _RPT)])

        @pl.when(s == 0)
        def _():
            pltpu.sync_copy(z128_hbm.at[pl.ds(_TAIL_OFF, _TAIL)], acch.at[pl.ds(_TAIL_OFF, _TAIL)])
            if with_ea:
                pltpu.sync_copy(z32_hbm.at[pl.ds(_TAIL_OFF, _TAIL)], accea.at[pl.ds(_TAIL_OFF, _TAIL)])

        plsc.subcore_barrier()

        bufs = ((idx0, rows0, ss0, ea0, es0), (idx1, rows1, ss1, ea1, es1))

        def idx_load(i, idx):
            pltpu.async_copy(idx_hbm.at[wid, i], idx, isem)

        def idx_wait(i, idx):
            pltpu.make_async_copy(idx_hbm.at[wid, i], idx, isem).wait()

        def gather(idx, rows):
            pltpu.async_copy(h_hbm.at[idx.at[0]], rows, gsem)

        def gather_wait(idx, rows):
            pltpu.make_async_copy(h_hbm.at[idx.at[0]], rows, gsem).wait()

        def scat(idx, rows, sem):
            pltpu.async_copy(rows, acch.at[idx.at[1]], sem, add=True)

        def scat_wait(idx, rows, sem):
            pltpu.make_async_copy(rows, acch.at[idx.at[1]], sem).wait()

        def ea_load(i, ea):
            pltpu.async_copy(ea_hbm.at[wid, i], ea, easem)

        def ea_wait(i, ea):
            pltpu.make_async_copy(ea_hbm.at[wid, i], ea, easem).wait()

        def ea_scat(idx, ea, sem):
            pltpu.async_copy(ea, accea.at[idx.at[1]], sem, add=True)

        def ea_scat_wait(idx, ea, sem):
            pltpu.make_async_copy(ea, accea.at[idx.at[1]], sem).wait()

        def phase(i, p, j=None, first_pair=False):
            idx, rows, ss, ea, es = bufs[p]
            idxq, rowsq, ssq, eaq, esq = bufs[1 - p]
            gather_wait(idx, rows)
            if with_ea:
                ea_wait(i, ea)
            scat(idx, rows, ss)
            if with_ea:
                ea_scat(idx, ea, es)

            def drain_prev():
                scat_wait(idxq, rowsq, ssq)
                if with_ea:
                    ea_scat_wait(idxq, eaq, esq)

            if first_pair:
                pl.when(j >= 1)(drain_prev)
            else:
                drain_prev()

            idx_load(i + 1, idxq)
            idx_wait(i + 1, idxq)
            gather(idxq, rowsq)
            if with_ea:
                ea_load(i + 1, eaq)

        idx_load(0, idx0)
        idx_wait(0, idx0)
        gather(idx0, rows0)
        if with_ea:
            ea_load(0, ea0)

        def pair(j, carry):
            phase(2 * j, 0, j=j, first_pair=True)
            phase(2 * j + 1, 1)
            return carry

        lax.fori_loop(0, _PAIRS, pair, 0)

        it = _STEPS - 1
        gather_wait(idx0, rows0)
        if with_ea:
            ea_wait(it, ea0)
        scat(idx0, rows0, ss0)
        if with_ea:
            ea_scat(idx0, ea0, es0)
        scat_wait(idx1, rows1, ss1)
        scat_wait(idx0, rows0, ss0)
        if with_ea:
            ea_scat_wait(idx1, ea1, es1)
            ea_scat_wait(idx0, ea0, es0)

        plsc.subcore_barrier()
        pltpu.sync_copy(acch.at[pl.ds(rbase, _RPT)], outh_hbm.at[c, pl.ds(rbase, _RPT)])
        if with_ea:
            pltpu.sync_copy(accea.at[pl.ds(rbase, _RPT)], outea_hbm.at[c, pl.ds(rbase, _RPT)])

        @pl.when(s == 0)
        def _():
            pltpu.sync_copy(acch.at[pl.ds(_TAIL_OFF, _TAIL)], outh_hbm.at[c, pl.ds(_TAIL_OFF, _TAIL)])
            if with_ea:
                pltpu.sync_copy(accea.at[pl.ds(_TAIL_OFF, _TAIL)], outea_hbm.at[c, pl.ds(_TAIL_OFF, _TAIL)])

    return k


_sc_agg_first = _make_sc_agg(True)
_sc_agg = _make_sc_agg(False)


_R = 1000


def _tc_layer(h, aggA, aggB, eaA, eaB, W, p3):
    def body(h_r, aA_r, aB_r, eA_r, eB_r, W_r, p_r, o_r):
        hb = h_r[...]
        agg = aA_r[...] + aB_r[...] + hb
        ea = eA_r[...] + eB_r[...]
        deg = ea[:, 16:17] + 1.0
        eav = ea[:, 0:16] + 1.0
        Wf = W_r[...]
        b = p_r[0:1, :]
        gs = p_r[1:2, :] * _BN_S
        be = p_r[2:3, :]
        out = (jnp.dot(hb, Wf[0:128], preferred_element_type=jnp.float32) + b) * deg
        out = out + jnp.dot(agg, Wf[128:256], preferred_element_type=jnp.float32)
        out = out + jnp.dot(eav, Wf[256:272], preferred_element_type=jnp.float32)
        hn = jnp.maximum(out, 0.0) * gs + be
        o_r[...] = jnp.maximum(hn, 0.0)

    return pl.pallas_call(
        body,
        grid=(_N // _R,),
        in_specs=[
            pl.BlockSpec((_R, _D), lambda i: (i, 0)),
            pl.BlockSpec((_R, _D), lambda i: (i, 0)),
            pl.BlockSpec((_R, _D), lambda i: (i, 0)),
            pl.BlockSpec((_R, _EAP), lambda i: (i, 0)),
            pl.BlockSpec((_R, _EAP), lambda i: (i, 0)),
            pl.BlockSpec((2 * _D + 16, _D), lambda i: (0, 0)),
            pl.BlockSpec((3, _D), lambda i: (0, 0)),
        ],
        out_specs=pl.BlockSpec((_R, _D), lambda i: (i, 0)),
        out_shape=jax.ShapeDtypeStruct((_N, _D), jnp.float32),
    )(h, aggA, aggB, eaA, eaB, W, p3)


def _tc_head(h1, h2, wp, wm1, wm2, bias1, fc2_W, fc2_b):
    def body(h1_r, h2_r, wp_r, w1_r, w2_r, b1_r, fw2_r, fb2_r, o_r):
        h2b = h2_r[...].reshape(_G, _NPG, _D)
        pooled = jnp.sum(h2b, axis=1)
        emb1 = h1_r[...].reshape(_G, _NPG, _D)[:, 0, :]
        emb2 = h2b[:, 0, :]
        r = jnp.dot(pooled, wp_r[...], preferred_element_type=jnp.float32)
        r = r + jnp.dot(emb1, w1_r[...], preferred_element_type=jnp.float32)
        r = r + jnp.dot(emb2, w2_r[...], preferred_element_type=jnp.float32)
        r = jnp.maximum(r + b1_r[...], 0.0)
        o_r[...] = jnp.dot(r, fw2_r[...], preferred_element_type=jnp.float32) + fb2_r[...]

    return pl.pallas_call(
        body,
        out_shape=jax.ShapeDtypeStruct((_G, _NC), jnp.float32),
    )(h1, h2, wp, wm1, wm2, bias1, fc2_W, fc2_b)


def kernel(x, edge_attr, W1, b1, g1, be1, W2, b2, g2, be2,
           fc1_W, fc1_b, fc2_W, fc2_b, edge_index, batch):
    idx4 = jnp.stack([edge_index[0].reshape(_NW, _STEPS, _C),
                      edge_index[1].reshape(_NW, _STEPS, _C)], axis=2)
    ea4 = jnp.concatenate(
        [edge_attr,
         jnp.ones((_E, 1), jnp.float32),
         jnp.zeros((_E, _EAP - 17), jnp.float32)],
        axis=1).reshape(_NW, _STEPS, _C, _EAP)
    z128 = jnp.zeros((_N, _D), jnp.float32)
    z32 = jnp.zeros((_N, _EAP), jnp.float32)

    aggh, aggea = _sc_agg_first(x, idx4, ea4, z128, z32)
    h1 = _tc_layer(x, aggh[0], aggh[1], aggea[0], aggea[1], W1,
                   jnp.stack([b1, g1, be1]))
    aggh2 = _sc_agg(h1, idx4, z128)
    h2 = _tc_layer(h1, aggh2[0], aggh2[1], aggea[0], aggea[1], W2,
                   jnp.stack([b2, g2, be2]))

    wp = fc1_W[0:128]
    wm1 = fc1_W[129:257]
    wm2 = fc1_W[257:385]
    bias1 = (fc1_b + fc1_W[128]).reshape(1, _MLP)
    return _tc_head(h1, h2, wp, wm1, wm2, bias1, fc2_W, fc2_b.reshape(1, _NC))


# speedup vs baseline: 11.1078x; 1.2048x over previous
import functools

import numpy as np
import jax
import jax.numpy as jnp
from jax import lax
from jax.experimental import pallas as pl
from jax.experimental.pallas import tpu as pltpu
from jax.experimental.pallas import tpu_sc as plsc

_N = 10000
_E = 320000
_D = 128
_DE = 16
_G = 250
_NPG = 40
_MLP = 256
_NC = 10
_BN_S = float(1.0 / np.sqrt(1.0 + 1e-5))
_DW = 8

_NW = 32
_PER_W = _E // _NW
_C = 80
_STEPS = _PER_W // _C
_UNROLL = 6
_LOOPN = (_STEPS - 5) // _UNROLL
_RPT = 624
_TAIL = _N - 16 * _RPT
_TAIL_OFF = 16 * _RPT

_mesh = plsc.VectorSubcoreMesh(core_axis_name="c", subcore_axis_name="s")


def _make_sc_agg(with_ea):
    outh_t = jax.ShapeDtypeStruct((2, _N, _D), jnp.float32)
    if with_ea:
        out_type = [outh_t,
                    jax.ShapeDtypeStruct((2, _N, _DE), jnp.float32),
                    jax.ShapeDtypeStruct((2, _N, _DW), jnp.float32)]
    else:
        out_type = outh_t
    scratch = [
        pltpu.VMEM((2, _C), jnp.int32),
        pltpu.VMEM((2, _C), jnp.int32),
        pltpu.VMEM((2, _C), jnp.int32),
        pltpu.VMEM((_C, _D), jnp.float32),
        pltpu.VMEM((_C, _D), jnp.float32),
        pltpu.VMEM_SHARED((_N, _D), jnp.float32),
        pltpu.SemaphoreType.DMA,
        pltpu.SemaphoreType.DMA,
        pltpu.SemaphoreType.DMA,
        pltpu.SemaphoreType.DMA,
    ]
    if with_ea:
        scratch += [
            pltpu.VMEM((_C, _DE), jnp.float32),
            pltpu.VMEM((_C, _DE), jnp.float32),
            pltpu.VMEM((_C, _DW), jnp.float32),
            pltpu.VMEM_SHARED((_N, _DE), jnp.float32),
            pltpu.VMEM_SHARED((_N, _DW), jnp.float32),
            pltpu.SemaphoreType.DMA,
            pltpu.SemaphoreType.DMA,
            pltpu.SemaphoreType.DMA,
            pltpu.SemaphoreType.DMA,
            pltpu.SemaphoreType.DMA,
        ]

    @functools.partial(
        pl.kernel,
        mesh=_mesh,
        compiler_params=pltpu.CompilerParams(use_tc_tiling_on_sc=False),
        out_type=out_type,
        scratch_types=scratch,
    )
    def k(*refs):
        if with_ea:
            (h_hbm, idx_hbm, ea_hbm, ones_hbm, z128_hbm, z16_hbm, z1_hbm,
             outh_hbm, outea_hbm, outdeg_hbm,
             idx0, idx1, idx2, rows0, rows1, acch, isem, gsem, ss0, ss1,
             eab0, eab1, dones, accea, accdeg, easem, es0, es1, ds0, ds1) = refs
        else:
            (h_hbm, idx_hbm, z128_hbm,
             outh_hbm,
             idx0, idx1, idx2, rows0, rows1, acch, isem, gsem, ss0, ss1) = refs
            eab0 = eab1 = dones = accea = accdeg = easem = None
            es0 = es1 = ds0 = ds1 = None

        c = lax.axis_index("c")
        s = lax.axis_index("s")
        wid = s * 2 + c
        rbase = s * _RPT
        ebase = wid * _PER_W

        idxb = (idx0, idx1, idx2)
        rowsb = (rows0, rows1)
        ssb = (ss0, ss1)
        eabb = (eab0, eab1)
        esb = (es0, es1)
        dsb = (ds0, ds1)

        pltpu.sync_copy(z128_hbm.at[pl.ds(rbase, _RPT)], acch.at[pl.ds(rbase, _RPT)])
        if with_ea:
            pltpu.sync_copy(z16_hbm.at[pl.ds(rbase, _RPT)], accea.at[pl.ds(rbase, _RPT)])
            pltpu.sync_copy(z1_hbm.at[pl.ds(rbase, _RPT)], accdeg.at[pl.ds(rbase, _RPT)])
            pltpu.sync_copy(ones_hbm, dones)

        @pl.when(s == 0)
        def _():
            pltpu.sync_copy(z128_hbm.at[pl.ds(_TAIL_OFF, _TAIL)], acch.at[pl.ds(_TAIL_OFF, _TAIL)])
            if with_ea:
                pltpu.sync_copy(z16_hbm.at[pl.ds(_TAIL_OFF, _TAIL)], accea.at[pl.ds(_TAIL_OFF, _TAIL)])
                pltpu.sync_copy(z1_hbm.at[pl.ds(_TAIL_OFF, _TAIL)], accdeg.at[pl.ds(_TAIL_OFF, _TAIL)])

        plsc.subcore_barrier()

        def idx_load(i, b):
            pltpu.async_copy(idx_hbm.at[wid, i], idxb[b], isem)

        def idx_wait(i, b):
            pltpu.make_async_copy(idx_hbm.at[wid, i], idxb[b], isem).wait()

        def gather(ib, rb):
            pltpu.async_copy(h_hbm.at[idxb[ib].at[0]], rowsb[rb], gsem)

        def gather_wait(ib, rb):
            pltpu.make_async_copy(h_hbm.at[idxb[ib].at[0]], rowsb[rb], gsem).wait()

        def ea_load(i, b):
            pltpu.async_copy(ea_hbm.at[pl.ds(ebase + i * _C, _C)], eabb[b], easem)

        def ea_wait(i, b):
            pltpu.make_async_copy(ea_hbm.at[pl.ds(ebase + i * _C, _C)], eabb[b], easem).wait()

        def scats(ib, p):
            dst = idxb[ib].at[1]
            pltpu.async_copy(rowsb[p], acch.at[dst], ssb[p], add=True)
            if with_ea:
                pltpu.async_copy(eabb[p], accea.at[dst], esb[p], add=True)
                pltpu.async_copy(dones, accdeg.at[dst], dsb[p], add=True)

        def scats_drain(ib, p):
            dst = idxb[ib].at[1]
            pltpu.make_async_copy(rowsb[p], acch.at[dst], ssb[p]).wait()
            if with_ea:
                pltpu.make_async_copy(eabb[p], accea.at[dst], esb[p]).wait()
                pltpu.make_async_copy(dones, accdeg.at[dst], dsb[p]).wait()

        def phase(i, k_, j=None, last=False, skip_idx2=False):
            p = k_ % 2
            q = (k_ + 1) % 2
            ib = k_ % 3
            ibn = (k_ + 1) % 3
            ibp = (k_ + 2) % 3
            gather_wait(ib, p)
            if with_ea:
                ea_wait(i, p)
            scats(ib, p)

            def drain_prev():
                scats_drain(ibp, q)

            if j is None:
                drain_prev()
            else:
                pl.when(j >= 1)(drain_prev)

            if last:
                return
            if not skip_idx2:
                idx_load(i + 2, ibp)
            idx_wait(i + 1, ibn)
            gather(ibn, q)
            if with_ea:
                ea_load(i + 1, q)

        idx_load(0, 0)
        idx_wait(0, 0)
        gather(0, 0)
        if with_ea:
            ea_load(0, 0)
        idx_load(1, 1)

        def body(j, carry):
            i6 = 6 * j
            phase(i6 + 0, 0, j=j)
            phase(i6 + 1, 1)
            phase(i6 + 2, 2)
            phase(i6 + 3, 3)
            phase(i6 + 4, 4)
            phase(i6 + 5, 5)
            return carry

        lax.fori_loop(0, _LOOPN, body, 0)

        phase(120, 0)
        phase(121, 1)
        phase(122, 2)
        phase(123, 3, skip_idx2=True)
        phase(124, 4, last=True)
        scats_drain(1, 0)

        plsc.subcore_barrier()
        pltpu.sync_copy(acch.at[pl.ds(rbase, _RPT)], outh_hbm.at[c, pl.ds(rbase, _RPT)])
        if with_ea:
            pltpu.sync_copy(accea.at[pl.ds(rbase, _RPT)], outea_hbm.at[c, pl.ds(rbase, _RPT)])
            pltpu.sync_copy(accdeg.at[pl.ds(rbase, _RPT)], outdeg_hbm.at[c, pl.ds(rbase, _RPT)])

        @pl.when(s == 0)
        def _():
            pltpu.sync_copy(acch.at[pl.ds(_TAIL_OFF, _TAIL)], outh_hbm.at[c, pl.ds(_TAIL_OFF, _TAIL)])
            if with_ea:
                pltpu.sync_copy(accea.at[pl.ds(_TAIL_OFF, _TAIL)], outea_hbm.at[c, pl.ds(_TAIL_OFF, _TAIL)])
                pltpu.sync_copy(accdeg.at[pl.ds(_TAIL_OFF, _TAIL)], outdeg_hbm.at[c, pl.ds(_TAIL_OFF, _TAIL)])

    return k


_sc_agg_first = _make_sc_agg(True)
_sc_agg = _make_sc_agg(False)


_R = 1000


def _tc_layer(h, aggA, aggB, eaA, eaB, dgA, dgB, W, p3):
    def body(h_r, aA_r, aB_r, eA_r, eB_r, dA_r, dB_r, W_r, p_r, o_r):
        hb = h_r[...]
        agg = aA_r[...] + aB_r[...] + hb
        eav = eA_r[...] + eB_r[...] + 1.0
        deg = dA_r[..., 0:1] + dB_r[..., 0:1] + 1.0
        Wf = W_r[...]
        b = p_r[0:1, :]
        gs = p_r[1:2, :] * _BN_S
        be = p_r[2:3, :]
        out = (jnp.dot(hb, Wf[0:128], preferred_element_type=jnp.float32) + b) * deg
        out = out + jnp.dot(agg, Wf[128:256], preferred_element_type=jnp.float32)
        out = out + jnp.dot(eav, Wf[256:272], preferred_element_type=jnp.float32)
        hn = jnp.maximum(out, 0.0) * gs + be
        o_r[...] = jnp.maximum(hn, 0.0)

    return pl.pallas_call(
        body,
        grid=(_N // _R,),
        in_specs=[
            pl.BlockSpec((_R, _D), lambda i: (i, 0)),
            pl.BlockSpec((_R, _D), lambda i: (i, 0)),
            pl.BlockSpec((_R, _D), lambda i: (i, 0)),
            pl.BlockSpec((_R, _DE), lambda i: (i, 0)),
            pl.BlockSpec((_R, _DE), lambda i: (i, 0)),
            pl.BlockSpec((_R, _DW), lambda i: (i, 0)),
            pl.BlockSpec((_R, _DW), lambda i: (i, 0)),
            pl.BlockSpec((2 * _D + _DE, _D), lambda i: (0, 0)),
            pl.BlockSpec((3, _D), lambda i: (0, 0)),
        ],
        out_specs=pl.BlockSpec((_R, _D), lambda i: (i, 0)),
        out_shape=jax.ShapeDtypeStruct((_N, _D), jnp.float32),
    )(h, aggA, aggB, eaA, eaB, dgA, dgB, W, p3)


def _tc_head(h1, h2, wp, wm1, wm2, bias1, fc2_W, fc2_b):
    def body(h1_r, h2_r, wp_r, w1_r, w2_r, b1_r, fw2_r, fb2_r, o_r):
        h2b = h2_r[...].reshape(_G, _NPG, _D)
        pooled = jnp.sum(h2b, axis=1)
        emb1 = h1_r[...].reshape(_G, _NPG, _D)[:, 0, :]
        emb2 = h2b[:, 0, :]
        r = jnp.dot(pooled, wp_r[...], preferred_element_type=jnp.float32)
        r = r + jnp.dot(emb1, w1_r[...], preferred_element_type=jnp.float32)
        r = r + jnp.dot(emb2, w2_r[...], preferred_element_type=jnp.float32)
        r = jnp.maximum(r + b1_r[...], 0.0)
        o_r[...] = jnp.dot(r, fw2_r[...], preferred_element_type=jnp.float32) + fb2_r[...]

    return pl.pallas_call(
        body,
        out_shape=jax.ShapeDtypeStruct((_G, _NC), jnp.float32),
    )(h1, h2, wp, wm1, wm2, bias1, fc2_W, fc2_b)


def kernel(x, edge_attr, W1, b1, g1, be1, W2, b2, g2, be2,
           fc1_W, fc1_b, fc2_W, fc2_b, edge_index, batch):
    idx4 = jnp.stack([edge_index[0].reshape(_NW, _STEPS, _C),
                      edge_index[1].reshape(_NW, _STEPS, _C)], axis=2)
    ones_c = jnp.concatenate([jnp.ones((_C, 1), jnp.float32), jnp.zeros((_C, _DW - 1), jnp.float32)], axis=1)
    z128 = jnp.zeros((_N, _D), jnp.float32)
    z16 = jnp.zeros((_N, _DE), jnp.float32)
    z1 = jnp.zeros((_N, _DW), jnp.float32)

    aggh, aggea, aggdg = _sc_agg_first(x, idx4, edge_attr, ones_c, z128, z16, z1)
    h1 = _tc_layer(x, aggh[0], aggh[1], aggea[0], aggea[1], aggdg[0], aggdg[1],
                   W1, jnp.stack([b1, g1, be1]))
    aggh2 = _sc_agg(h1, idx4, z128)
    h2 = _tc_layer(h1, aggh2[0], aggh2[1], aggea[0], aggea[1], aggdg[0], aggdg[1],
                   W2, jnp.stack([b2, g2, be2]))

    wp = fc1_W[0:128]
    wm1 = fc1_W[129:257]
    wm2 = fc1_W[257:385]
    bias1 = (fc1_b + fc1_W[128]).reshape(1, _MLP)
    return _tc_head(h1, h2, wp, wm1, wm2, bias1, fc2_W, fc2_b.reshape(1, _NC))


# speedup vs baseline: 12.5108x; 1.1263x over previous
import functools

import numpy as np
import jax
import jax.numpy as jnp
from jax import lax
from jax.experimental import pallas as pl
from jax.experimental.pallas import tpu as pltpu
from jax.experimental.pallas import tpu_sc as plsc

_N = 10000
_E = 320000
_D = 128
_DE = 16
_G = 250
_NPG = 40
_MLP = 256
_NC = 10
_BN_S = float(1.0 / np.sqrt(1.0 + 1e-5))
_DW = 8

_NW = 32
_PER_W = _E // _NW
_C1 = 112
_C2 = 128
_RPT = 624
_TAIL = _N - 16 * _RPT
_TAIL_OFF = 16 * _RPT

_mesh = plsc.VectorSubcoreMesh(core_axis_name="c", subcore_axis_name="s")


def _make_sc_agg(with_ea, C):
    STEPS = _PER_W // C
    CT = _PER_W - STEPS * C
    LOOPN = (STEPS - 5) // 6
    outh_t = jax.ShapeDtypeStruct((2, _N, _D), jnp.float32)
    if with_ea:
        out_type = [outh_t,
                    jax.ShapeDtypeStruct((2, _N, _DE), jnp.float32),
                    jax.ShapeDtypeStruct((2, _N, _DW), jnp.float32)]
    else:
        out_type = outh_t
    scratch = [
        pltpu.VMEM((2, C), jnp.int32),
        pltpu.VMEM((2, C), jnp.int32),
        pltpu.VMEM((2, C), jnp.int32),
        pltpu.VMEM((C, _D), jnp.float32),
        pltpu.VMEM((C, _D), jnp.float32),
        pltpu.VMEM_SHARED((_N, _D), jnp.float32),
        pltpu.SemaphoreType.DMA,
        pltpu.SemaphoreType.DMA,
        pltpu.SemaphoreType.DMA,
        pltpu.SemaphoreType.DMA,
    ]
    if with_ea:
        scratch += [
            pltpu.VMEM((C, _DE), jnp.float32),
            pltpu.VMEM((C, _DE), jnp.float32),
            pltpu.VMEM((C, _DW), jnp.float32),
            pltpu.VMEM_SHARED((_N, _DE), jnp.float32),
            pltpu.VMEM_SHARED((_N, _DW), jnp.float32),
            pltpu.SemaphoreType.DMA,
            pltpu.SemaphoreType.DMA,
            pltpu.SemaphoreType.DMA,
            pltpu.SemaphoreType.DMA,
            pltpu.SemaphoreType.DMA,
        ]
    scratch.append(pltpu.VMEM((2, CT), jnp.int32))

    @functools.partial(
        pl.kernel,
        mesh=_mesh,
        compiler_params=pltpu.CompilerParams(use_tc_tiling_on_sc=False),
        out_type=out_type,
        scratch_types=scratch,
    )
    def k(*refs):
        if with_ea:
            (h_hbm, idx_hbm, ea_hbm, ones_hbm, z128_hbm, z16_hbm, z1_hbm, idxt_hbm,
             outh_hbm, outea_hbm, outdeg_hbm,
             idx0, idx1, idx2, rows0, rows1, acch, isem, gsem, ss0, ss1,
             eab0, eab1, dones, accea, accdeg, easem, es0, es1, ds0, ds1, idxt) = refs
        else:
            (h_hbm, idx_hbm, z128_hbm, idxt_hbm,
             outh_hbm,
             idx0, idx1, idx2, rows0, rows1, acch, isem, gsem, ss0, ss1, idxt) = refs
            eab0 = eab1 = dones = accea = accdeg = easem = None
            es0 = es1 = ds0 = ds1 = None

        c = lax.axis_index("c")
        s = lax.axis_index("s")
        wid = s * 2 + c
        rbase = s * _RPT
        ebase = wid * _PER_W

        idxb = (idx0, idx1, idx2)
        rowsb = (rows0, rows1)
        ssb = (ss0, ss1)
        eabb = (eab0, eab1)
        esb = (es0, es1)
        dsb = (ds0, ds1)

        pltpu.sync_copy(z128_hbm.at[pl.ds(rbase, _RPT)], acch.at[pl.ds(rbase, _RPT)])
        if with_ea:
            pltpu.sync_copy(z16_hbm.at[pl.ds(rbase, _RPT)], accea.at[pl.ds(rbase, _RPT)])
            pltpu.sync_copy(z1_hbm.at[pl.ds(rbase, _RPT)], accdeg.at[pl.ds(rbase, _RPT)])
            pltpu.sync_copy(ones_hbm, dones)

        @pl.when(s == 0)
        def _():
            pltpu.sync_copy(z128_hbm.at[pl.ds(_TAIL_OFF, _TAIL)], acch.at[pl.ds(_TAIL_OFF, _TAIL)])
            if with_ea:
                pltpu.sync_copy(z16_hbm.at[pl.ds(_TAIL_OFF, _TAIL)], accea.at[pl.ds(_TAIL_OFF, _TAIL)])
                pltpu.sync_copy(z1_hbm.at[pl.ds(_TAIL_OFF, _TAIL)], accdeg.at[pl.ds(_TAIL_OFF, _TAIL)])

        plsc.subcore_barrier()

        def idx_load(i, b):
            pltpu.async_copy(idx_hbm.at[wid, i], idxb[b], isem)

        def idx_wait(i, b):
            pltpu.make_async_copy(idx_hbm.at[wid, i], idxb[b], isem).wait()

        def gather(ib, rb):
            pltpu.async_copy(h_hbm.at[idxb[ib].at[0]], rowsb[rb], gsem)

        def gather_wait(ib, rb):
            pltpu.make_async_copy(h_hbm.at[idxb[ib].at[0]], rowsb[rb], gsem).wait()

        def ea_load(i, b):
            pltpu.async_copy(ea_hbm.at[pl.ds(ebase + i * C, C)], eabb[b], easem)

        def ea_wait(i, b):
            pltpu.make_async_copy(ea_hbm.at[pl.ds(ebase + i * C, C)], eabb[b], easem).wait()

        def scats(ib, p):
            dst = idxb[ib].at[1]
            pltpu.async_copy(rowsb[p], acch.at[dst], ssb[p], add=True)
            if with_ea:
                pltpu.async_copy(eabb[p], accea.at[dst], esb[p], add=True)
                pltpu.async_copy(dones, accdeg.at[dst], dsb[p], add=True)

        def scats_drain(ib, p):
            dst = idxb[ib].at[1]
            pltpu.make_async_copy(rowsb[p], acch.at[dst], ssb[p]).wait()
            if with_ea:
                pltpu.make_async_copy(eabb[p], accea.at[dst], esb[p]).wait()
                pltpu.make_async_copy(dones, accdeg.at[dst], dsb[p]).wait()

        def phase(i, k_, j=None, last=False, skip_idx2=False):
            p = k_ % 2
            q = (k_ + 1) % 2
            ib = k_ % 3
            ibn = (k_ + 1) % 3
            ibp = (k_ + 2) % 3
            gather_wait(ib, p)
            if with_ea:
                ea_wait(i, p)
            scats(ib, p)

            def drain_prev():
                scats_drain(ibp, q)

            if j is None:
                drain_prev()
            else:
                pl.when(j >= 1)(drain_prev)

            if last:
                return
            if not skip_idx2:
                idx_load(i + 2, ibp)
            idx_wait(i + 1, ibn)
            gather(ibn, q)
            if with_ea:
                ea_load(i + 1, q)

        idx_load(0, 0)
        idx_wait(0, 0)
        gather(0, 0)
        if with_ea:
            ea_load(0, 0)
        idx_load(1, 1)

        def body(j, carry):
            i6 = 6 * j
            phase(i6 + 0, 0, j=j)
            phase(i6 + 1, 1)
            phase(i6 + 2, 2)
            phase(i6 + 3, 3)
            phase(i6 + 4, 4)
            phase(i6 + 5, 5)
            return carry

        lax.fori_loop(0, LOOPN, body, 0)

        for i in range(6 * LOOPN, STEPS):
            phase(i, i % 6, last=(i == STEPS - 1), skip_idx2=(i + 2 >= STEPS))
        scats_drain((STEPS - 1) % 3, (STEPS - 1) % 2)

        if CT:
            tbase = ebase + STEPS * C
            pltpu.sync_copy(idxt_hbm.at[wid], idxt)
            pltpu.async_copy(h_hbm.at[idxt.at[0]], rows0.at[pl.ds(0, CT)], gsem)
            pltpu.make_async_copy(h_hbm.at[idxt.at[0]], rows0.at[pl.ds(0, CT)], gsem).wait()
            pltpu.async_copy(rows0.at[pl.ds(0, CT)], acch.at[idxt.at[1]], ss0, add=True)
            if with_ea:
                pltpu.sync_copy(ea_hbm.at[pl.ds(tbase, CT)], eab0.at[pl.ds(0, CT)])
                pltpu.async_copy(eab0.at[pl.ds(0, CT)], accea.at[idxt.at[1]], es0, add=True)
                pltpu.async_copy(dones.at[pl.ds(0, CT)], accdeg.at[idxt.at[1]], ds0, add=True)
            pltpu.make_async_copy(rows0.at[pl.ds(0, CT)], acch.at[idxt.at[1]], ss0).wait()
            if with_ea:
                pltpu.make_async_copy(eab0.at[pl.ds(0, CT)], accea.at[idxt.at[1]], es0).wait()
                pltpu.make_async_copy(dones.at[pl.ds(0, CT)], accdeg.at[idxt.at[1]], ds0).wait()

        plsc.subcore_barrier()
        pltpu.sync_copy(acch.at[pl.ds(rbase, _RPT)], outh_hbm.at[c, pl.ds(rbase, _RPT)])
        if with_ea:
            pltpu.sync_copy(accea.at[pl.ds(rbase, _RPT)], outea_hbm.at[c, pl.ds(rbase, _RPT)])
            pltpu.sync_copy(accdeg.at[pl.ds(rbase, _RPT)], outdeg_hbm.at[c, pl.ds(rbase, _RPT)])

        @pl.when(s == 0)
        def _():
            pltpu.sync_copy(acch.at[pl.ds(_TAIL_OFF, _TAIL)], outh_hbm.at[c, pl.ds(_TAIL_OFF, _TAIL)])
            if with_ea:
                pltpu.sync_copy(accea.at[pl.ds(_TAIL_OFF, _TAIL)], outea_hbm.at[c, pl.ds(_TAIL_OFF, _TAIL)])
                pltpu.sync_copy(accdeg.at[pl.ds(_TAIL_OFF, _TAIL)], outdeg_hbm.at[c, pl.ds(_TAIL_OFF, _TAIL)])

    return k


_sc_agg_first = _make_sc_agg(True, _C1)
_sc_agg = _make_sc_agg(False, _C2)


_R = 1000


def _tc_layer(h, aggA, aggB, eaA, eaB, dgA, dgB, W, p3):
    def body(h_r, aA_r, aB_r, eA_r, eB_r, dA_r, dB_r, W_r, p_r, o_r):
        hb = h_r[...]
        agg = aA_r[...] + aB_r[...] + hb
        eav = eA_r[...] + eB_r[...] + 1.0
        deg = dA_r[..., 0:1] + dB_r[..., 0:1] + 1.0
        Wf = W_r[...]
        b = p_r[0:1, :]
        gs = p_r[1:2, :] * _BN_S
        be = p_r[2:3, :]
        out = (jnp.dot(hb, Wf[0:128], preferred_element_type=jnp.float32) + b) * deg
        out = out + jnp.dot(agg, Wf[128:256], preferred_element_type=jnp.float32)
        out = out + jnp.dot(eav, Wf[256:272], preferred_element_type=jnp.float32)
        hn = jnp.maximum(out, 0.0) * gs + be
        o_r[...] = jnp.maximum(hn, 0.0)

    return pl.pallas_call(
        body,
        grid=(_N // _R,),
        in_specs=[
            pl.BlockSpec((_R, _D), lambda i: (i, 0)),
            pl.BlockSpec((_R, _D), lambda i: (i, 0)),
            pl.BlockSpec((_R, _D), lambda i: (i, 0)),
            pl.BlockSpec((_R, _DE), lambda i: (i, 0)),
            pl.BlockSpec((_R, _DE), lambda i: (i, 0)),
            pl.BlockSpec((_R, _DW), lambda i: (i, 0)),
            pl.BlockSpec((_R, _DW), lambda i: (i, 0)),
            pl.BlockSpec((2 * _D + _DE, _D), lambda i: (0, 0)),
            pl.BlockSpec((3, _D), lambda i: (0, 0)),
        ],
        out_specs=pl.BlockSpec((_R, _D), lambda i: (i, 0)),
        out_shape=jax.ShapeDtypeStruct((_N, _D), jnp.float32),
    )(h, aggA, aggB, eaA, eaB, dgA, dgB, W, p3)


def _tc_head(h1, h2, wp, wm1, wm2, bias1, fc2_W, fc2_b):
    def body(h1_r, h2_r, wp_r, w1_r, w2_r, b1_r, fw2_r, fb2_r, o_r):
        h2b = h2_r[...].reshape(_G, _NPG, _D)
        pooled = jnp.sum(h2b, axis=1)
        emb1 = h1_r[...].reshape(_G, _NPG, _D)[:, 0, :]
        emb2 = h2b[:, 0, :]
        r = jnp.dot(pooled, wp_r[...], preferred_element_type=jnp.float32)
        r = r + jnp.dot(emb1, w1_r[...], preferred_element_type=jnp.float32)
        r = r + jnp.dot(emb2, w2_r[...], preferred_element_type=jnp.float32)
        r = jnp.maximum(r + b1_r[...], 0.0)
        o_r[...] = jnp.dot(r, fw2_r[...], preferred_element_type=jnp.float32) + fb2_r[...]

    return pl.pallas_call(
        body,
        out_shape=jax.ShapeDtypeStruct((_G, _NC), jnp.float32),
    )(h1, h2, wp, wm1, wm2, bias1, fc2_W, fc2_b)


def kernel(x, edge_attr, W1, b1, g1, be1, W2, b2, g2, be2,
           fc1_W, fc1_b, fc2_W, fc2_b, edge_index, batch):
    ei3 = edge_index.reshape(2, _NW, _PER_W)
    s1 = _PER_W // _C1
    s2 = _PER_W // _C2
    idx4a = ei3[:, :, :s1 * _C1].reshape(2, _NW, s1, _C1).transpose(1, 2, 0, 3)
    idxta = ei3[:, :, s1 * _C1:].transpose(1, 0, 2)
    idx4b = ei3[:, :, :s2 * _C2].reshape(2, _NW, s2, _C2).transpose(1, 2, 0, 3)
    idxtb = ei3[:, :, s2 * _C2:].transpose(1, 0, 2)
    ones_c = jnp.concatenate([jnp.ones((_C1, 1), jnp.float32), jnp.zeros((_C1, _DW - 1), jnp.float32)], axis=1)
    z128 = jnp.zeros((_N, _D), jnp.float32)
    z16 = jnp.zeros((_N, _DE), jnp.float32)
    z1 = jnp.zeros((_N, _DW), jnp.float32)

    aggh, aggea, aggdg = _sc_agg_first(x, idx4a, edge_attr, ones_c, z128, z16, z1, idxta)
    h1 = _tc_layer(x, aggh[0], aggh[1], aggea[0], aggea[1], aggdg[0], aggdg[1],
                   W1, jnp.stack([b1, g1, be1]))
    aggh2 = _sc_agg(h1, idx4b, z128, idxtb)
    h2 = _tc_layer(h1, aggh2[0], aggh2[1], aggea[0], aggea[1], aggdg[0], aggdg[1],
                   W2, jnp.stack([b2, g2, be2]))

    wp = fc1_W[0:128]
    wm1 = fc1_W[129:257]
    wm2 = fc1_W[257:385]
    bias1 = (fc1_b + fc1_W[128]).reshape(1, _MLP)
    return _tc_head(h1, h2, wp, wm1, wm2, bias1, fc2_W, fc2_b.reshape(1, _NC))


# speedup vs baseline: 12.5219x; 1.0009x over previous
import functools

import numpy as np
import jax
import jax.numpy as jnp
from jax import lax
from jax.experimental import pallas as pl
from jax.experimental.pallas import tpu as pltpu
from jax.experimental.pallas import tpu_sc as plsc

_N = 10000
_E = 320000
_D = 128
_DE = 16
_G = 250
_NPG = 40
_MLP = 256
_NC = 10
_BN_S = float(1.0 / np.sqrt(1.0 + 1e-5))
_DW = 8

_NW = 32
_PER_W = _E // _NW
_C1 = 112
_C2 = 128
_RPT = 624
_TAIL = _N - 16 * _RPT
_TAIL_OFF = 16 * _RPT

_mesh = plsc.VectorSubcoreMesh(core_axis_name="c", subcore_axis_name="s")


def _make_sc_agg(with_ea, C, deep=False):
    STEPS = _PER_W // C
    CT = _PER_W - STEPS * C
    LOOPN = (STEPS - 5) // 6
    outh_t = jax.ShapeDtypeStruct((2, _N, _D), jnp.float32)
    if with_ea:
        out_type = [outh_t,
                    jax.ShapeDtypeStruct((2, _N, _DE), jnp.float32),
                    jax.ShapeDtypeStruct((2, _N, _DW), jnp.float32)]
    else:
        out_type = outh_t
    scratch = [
        pltpu.VMEM((2, C), jnp.int32),
        pltpu.VMEM((2, C), jnp.int32),
        pltpu.VMEM((2, C), jnp.int32),
        pltpu.VMEM((C, _D), jnp.float32),
        pltpu.VMEM((C, _D), jnp.float32),
        pltpu.VMEM_SHARED((_N, _D), jnp.float32),
        pltpu.SemaphoreType.DMA,
        pltpu.SemaphoreType.DMA,
        pltpu.SemaphoreType.DMA,
        pltpu.SemaphoreType.DMA,
    ]
    if deep:
        scratch += [pltpu.VMEM((C, _D), jnp.float32),
                    pltpu.SemaphoreType.DMA]
    if with_ea:
        scratch += [
            pltpu.VMEM((C, _DE), jnp.float32),
            pltpu.VMEM((C, _DE), jnp.float32),
            pltpu.VMEM((C, _DW), jnp.float32),
            pltpu.VMEM_SHARED((_N, _DE), jnp.float32),
            pltpu.VMEM_SHARED((_N, _DW), jnp.float32),
            pltpu.SemaphoreType.DMA,
            pltpu.SemaphoreType.DMA,
            pltpu.SemaphoreType.DMA,
            pltpu.SemaphoreType.DMA,
            pltpu.SemaphoreType.DMA,
        ]
    scratch.append(pltpu.VMEM((2, CT), jnp.int32))

    @functools.partial(
        pl.kernel,
        mesh=_mesh,
        compiler_params=pltpu.CompilerParams(use_tc_tiling_on_sc=False),
        out_type=out_type,
        scratch_types=scratch,
    )
    def k(*refs):
        if with_ea:
            (h_hbm, idx_hbm, ea_hbm, ones_hbm, z128_hbm, z16_hbm, z1_hbm, idxt_hbm,
             outh_hbm, outea_hbm, outdeg_hbm,
             idx0, idx1, idx2, rows0, rows1, acch, isem, gsem, ss0, ss1,
             eab0, eab1, dones, accea, accdeg, easem, es0, es1, ds0, ds1, idxt) = refs
        else:
            if deep:
                (h_hbm, idx_hbm, z128_hbm, idxt_hbm,
                 outh_hbm,
                 idx0, idx1, idx2, rows0, rows1, acch, isem, gsem, ss0, ss1,
                 rows2, ss2, idxt) = refs
            else:
                (h_hbm, idx_hbm, z128_hbm, idxt_hbm,
                 outh_hbm,
                 idx0, idx1, idx2, rows0, rows1, acch, isem, gsem, ss0, ss1) = refs[:15]
                idxt = refs[15]
                rows2 = ss2 = None
            eab0 = eab1 = dones = accea = accdeg = easem = None
            es0 = es1 = ds0 = ds1 = None

        c = lax.axis_index("c")
        s = lax.axis_index("s")
        wid = s * 2 + c
        rbase = s * _RPT
        ebase = wid * _PER_W

        idxb = (idx0, idx1, idx2)
        rowsb = (rows0, rows1, rows2) if deep else (rows0, rows1)
        ssb = (ss0, ss1, ss2) if deep else (ss0, ss1)
        eabb = (eab0, eab1)
        esb = (es0, es1)
        dsb = (ds0, ds1)

        pltpu.sync_copy(z128_hbm.at[pl.ds(rbase, _RPT)], acch.at[pl.ds(rbase, _RPT)])
        if with_ea:
            pltpu.sync_copy(z16_hbm.at[pl.ds(rbase, _RPT)], accea.at[pl.ds(rbase, _RPT)])
            pltpu.sync_copy(z1_hbm.at[pl.ds(rbase, _RPT)], accdeg.at[pl.ds(rbase, _RPT)])
            pltpu.sync_copy(ones_hbm, dones)

        @pl.when(s == 0)
        def _():
            pltpu.sync_copy(z128_hbm.at[pl.ds(_TAIL_OFF, _TAIL)], acch.at[pl.ds(_TAIL_OFF, _TAIL)])
            if with_ea:
                pltpu.sync_copy(z16_hbm.at[pl.ds(_TAIL_OFF, _TAIL)], accea.at[pl.ds(_TAIL_OFF, _TAIL)])
                pltpu.sync_copy(z1_hbm.at[pl.ds(_TAIL_OFF, _TAIL)], accdeg.at[pl.ds(_TAIL_OFF, _TAIL)])

        plsc.subcore_barrier()

        def idx_load(i, b):
            pltpu.async_copy(idx_hbm.at[wid, i], idxb[b], isem)

        def idx_wait(i, b):
            pltpu.make_async_copy(idx_hbm.at[wid, i], idxb[b], isem).wait()

        def gather(ib, rb):
            pltpu.async_copy(h_hbm.at[idxb[ib].at[0]], rowsb[rb], gsem)

        def gather_wait(ib, rb):
            pltpu.make_async_copy(h_hbm.at[idxb[ib].at[0]], rowsb[rb], gsem).wait()

        def ea_load(i, b):
            pltpu.async_copy(ea_hbm.at[pl.ds(ebase + i * C, C)], eabb[b], easem)

        def ea_wait(i, b):
            pltpu.make_async_copy(ea_hbm.at[pl.ds(ebase + i * C, C)], eabb[b], easem).wait()

        def scats(ib, p):
            dst = idxb[ib].at[1]
            pltpu.async_copy(rowsb[p], acch.at[dst], ssb[p], add=True)
            if with_ea:
                pltpu.async_copy(eabb[p], accea.at[dst], esb[p], add=True)
                pltpu.async_copy(dones, accdeg.at[dst], dsb[p], add=True)

        def scats_drain(ib, p):
            dst = idxb[ib].at[1]
            pltpu.make_async_copy(rowsb[p], acch.at[dst], ssb[p]).wait()
            if with_ea:
                pltpu.make_async_copy(eabb[p], accea.at[dst], esb[p]).wait()
                pltpu.make_async_copy(dones, accdeg.at[dst], dsb[p]).wait()

        def phase(i, k_, j=None, last=False, skip_idx2=False):
            p = k_ % 2
            q = (k_ + 1) % 2
            ib = k_ % 3
            ibn = (k_ + 1) % 3
            ibp = (k_ + 2) % 3
            gather_wait(ib, p)
            if with_ea:
                ea_wait(i, p)
            scats(ib, p)

            def drain_prev():
                scats_drain(ibp, q)

            if j is None:
                drain_prev()
            else:
                pl.when(j >= 1)(drain_prev)

            if last:
                return
            if not skip_idx2:
                idx_load(i + 2, ibp)
            idx_wait(i + 1, ibn)
            gather(ibn, q)
            if with_ea:
                ea_load(i + 1, q)

        def phase_deep(i, k_, j=None, last=False, skip_idx2=False):
            r = k_ % 3
            rn = (k_ + 1) % 3
            rp = (k_ + 2) % 3
            gather_wait(r, r)
            if not last:
                idx_wait(i + 1, rn)
                gather(rn, rn)
            pltpu.async_copy(rowsb[r], acch.at[idxb[r].at[1]], ssb[r], add=True)

            def drain_prev():
                pltpu.make_async_copy(rowsb[rp], acch.at[idxb[rp].at[1]], ssb[rp]).wait()

            if j is None:
                drain_prev()
            else:
                pl.when(j >= 1)(drain_prev)
            if not (last or skip_idx2):
                idx_load(i + 2, rp)

        phase_fn = phase_deep if deep else phase

        idx_load(0, 0)
        idx_wait(0, 0)
        gather(0, 0)
        if with_ea:
            ea_load(0, 0)
        idx_load(1, 1)

        def body(j, carry):
            i6 = 6 * j
            phase_fn(i6 + 0, 0, j=j)
            phase_fn(i6 + 1, 1)
            phase_fn(i6 + 2, 2)
            phase_fn(i6 + 3, 3)
            phase_fn(i6 + 4, 4)
            phase_fn(i6 + 5, 5)
            return carry

        lax.fori_loop(0, LOOPN, body, 0)

        for i in range(6 * LOOPN, STEPS):
            phase_fn(i, i % 6, last=(i == STEPS - 1), skip_idx2=(i + 2 >= STEPS))
        if deep:
            scats_drain((STEPS - 1) % 3, (STEPS - 1) % 3)
        else:
            scats_drain((STEPS - 1) % 3, (STEPS - 1) % 2)

        if CT:
            tbase = ebase + STEPS * C
            pltpu.sync_copy(idxt_hbm.at[wid], idxt)
            pltpu.async_copy(h_hbm.at[idxt.at[0]], rows0.at[pl.ds(0, CT)], gsem)
            pltpu.make_async_copy(h_hbm.at[idxt.at[0]], rows0.at[pl.ds(0, CT)], gsem).wait()
            pltpu.async_copy(rows0.at[pl.ds(0, CT)], acch.at[idxt.at[1]], ss0, add=True)
            if with_ea:
                pltpu.sync_copy(ea_hbm.at[pl.ds(tbase, CT)], eab0.at[pl.ds(0, CT)])
                pltpu.async_copy(eab0.at[pl.ds(0, CT)], accea.at[idxt.at[1]], es0, add=True)
                pltpu.async_copy(dones.at[pl.ds(0, CT)], accdeg.at[idxt.at[1]], ds0, add=True)
            pltpu.make_async_copy(rows0.at[pl.ds(0, CT)], acch.at[idxt.at[1]], ss0).wait()
            if with_ea:
                pltpu.make_async_copy(eab0.at[pl.ds(0, CT)], accea.at[idxt.at[1]], es0).wait()
                pltpu.make_async_copy(dones.at[pl.ds(0, CT)], accdeg.at[idxt.at[1]], ds0).wait()

        plsc.subcore_barrier()
        pltpu.sync_copy(acch.at[pl.ds(rbase, _RPT)], outh_hbm.at[c, pl.ds(rbase, _RPT)])
        if with_ea:
            pltpu.sync_copy(accea.at[pl.ds(rbase, _RPT)], outea_hbm.at[c, pl.ds(rbase, _RPT)])
            pltpu.sync_copy(accdeg.at[pl.ds(rbase, _RPT)], outdeg_hbm.at[c, pl.ds(rbase, _RPT)])

        @pl.when(s == 0)
        def _():
            pltpu.sync_copy(acch.at[pl.ds(_TAIL_OFF, _TAIL)], outh_hbm.at[c, pl.ds(_TAIL_OFF, _TAIL)])
            if with_ea:
                pltpu.sync_copy(accea.at[pl.ds(_TAIL_OFF, _TAIL)], outea_hbm.at[c, pl.ds(_TAIL_OFF, _TAIL)])
                pltpu.sync_copy(accdeg.at[pl.ds(_TAIL_OFF, _TAIL)], outdeg_hbm.at[c, pl.ds(_TAIL_OFF, _TAIL)])

    return k


_sc_agg_first = _make_sc_agg(True, _C1)
_sc_agg = _make_sc_agg(False, _C2, deep=True)


_R = 1000


def _tc_layer(h, aggA, aggB, eaA, eaB, dgA, dgB, W, p3):
    def body(h_r, aA_r, aB_r, eA_r, eB_r, dA_r, dB_r, W_r, p_r, o_r):
        hb = h_r[...]
        agg = aA_r[...] + aB_r[...] + hb
        eav = eA_r[...] + eB_r[...] + 1.0
        deg = dA_r[..., 0:1] + dB_r[..., 0:1] + 1.0
        Wf = W_r[...]
        b = p_r[0:1, :]
        gs = p_r[1:2, :] * _BN_S
        be = p_r[2:3, :]
        out = (jnp.dot(hb, Wf[0:128], preferred_element_type=jnp.float32) + b) * deg
        out = out + jnp.dot(agg, Wf[128:256], preferred_element_type=jnp.float32)
        out = out + jnp.dot(eav, Wf[256:272], preferred_element_type=jnp.float32)
        hn = jnp.maximum(out, 0.0) * gs + be
        o_r[...] = jnp.maximum(hn, 0.0)

    return pl.pallas_call(
        body,
        grid=(_N // _R,),
        in_specs=[
            pl.BlockSpec((_R, _D), lambda i: (i, 0)),
            pl.BlockSpec((_R, _D), lambda i: (i, 0)),
            pl.BlockSpec((_R, _D), lambda i: (i, 0)),
            pl.BlockSpec((_R, _DE), lambda i: (i, 0)),
            pl.BlockSpec((_R, _DE), lambda i: (i, 0)),
            pl.BlockSpec((_R, _DW), lambda i: (i, 0)),
            pl.BlockSpec((_R, _DW), lambda i: (i, 0)),
            pl.BlockSpec((2 * _D + _DE, _D), lambda i: (0, 0)),
            pl.BlockSpec((3, _D), lambda i: (0, 0)),
        ],
        out_specs=pl.BlockSpec((_R, _D), lambda i: (i, 0)),
        out_shape=jax.ShapeDtypeStruct((_N, _D), jnp.float32),
    )(h, aggA, aggB, eaA, eaB, dgA, dgB, W, p3)


def _tc_head(h1, h2, wp, wm1, wm2, bias1, fc2_W, fc2_b):
    def body(h1_r, h2_r, wp_r, w1_r, w2_r, b1_r, fw2_r, fb2_r, o_r):
        h2b = h2_r[...].reshape(_G, _NPG, _D)
        pooled = jnp.sum(h2b, axis=1)
        emb1 = h1_r[...].reshape(_G, _NPG, _D)[:, 0, :]
        emb2 = h2b[:, 0, :]
        r = jnp.dot(pooled, wp_r[...], preferred_element_type=jnp.float32)
        r = r + jnp.dot(emb1, w1_r[...], preferred_element_type=jnp.float32)
        r = r + jnp.dot(emb2, w2_r[...], preferred_element_type=jnp.float32)
        r = jnp.maximum(r + b1_r[...], 0.0)
        o_r[...] = jnp.dot(r, fw2_r[...], preferred_element_type=jnp.float32) + fb2_r[...]

    return pl.pallas_call(
        body,
        out_shape=jax.ShapeDtypeStruct((_G, _NC), jnp.float32),
    )(h1, h2, wp, wm1, wm2, bias1, fc2_W, fc2_b)


def kernel(x, edge_attr, W1, b1, g1, be1, W2, b2, g2, be2,
           fc1_W, fc1_b, fc2_W, fc2_b, edge_index, batch):
    ei3 = edge_index.reshape(2, _NW, _PER_W)
    s1 = _PER_W // _C1
    s2 = _PER_W // _C2
    idx4a = ei3[:, :, :s1 * _C1].reshape(2, _NW, s1, _C1).transpose(1, 2, 0, 3)
    idxta = ei3[:, :, s1 * _C1:].transpose(1, 0, 2)
    idx4b = ei3[:, :, :s2 * _C2].reshape(2, _NW, s2, _C2).transpose(1, 2, 0, 3)
    idxtb = ei3[:, :, s2 * _C2:].transpose(1, 0, 2)
    ones_c = jnp.concatenate([jnp.ones((_C1, 1), jnp.float32), jnp.zeros((_C1, _DW - 1), jnp.float32)], axis=1)
    z128 = jnp.zeros((_N, _D), jnp.float32)
    z16 = jnp.zeros((_N, _DE), jnp.float32)
    z1 = jnp.zeros((_N, _DW), jnp.float32)

    aggh, aggea, aggdg = _sc_agg_first(x, idx4a, edge_attr, ones_c, z128, z16, z1, idxta)
    h1 = _tc_layer(x, aggh[0], aggh[1], aggea[0], aggea[1], aggdg[0], aggdg[1],
                   W1, jnp.stack([b1, g1, be1]))
    aggh2 = _sc_agg(h1, idx4b, z128, idxtb)
    h2 = _tc_layer(h1, aggh2[0], aggh2[1], aggea[0], aggea[1], aggdg[0], aggdg[1],
                   W2, jnp.stack([b2, g2, be2]))

    wp = fc1_W[0:128]
    wm1 = fc1_W[129:257]
    wm2 = fc1_W[257:385]
    bias1 = (fc1_b + fc1_W[128]).reshape(1, _MLP)
    return _tc_head(h1, h2, wp, wm1, wm2, bias1, fc2_W, fc2_b.reshape(1, _NC))


# speedup vs baseline: 12.6801x; 1.0126x over previous
import functools

import numpy as np
import jax
import jax.numpy as jnp
from jax import lax
from jax.experimental import pallas as pl
from jax.experimental.pallas import tpu as pltpu
from jax.experimental.pallas import tpu_sc as plsc

_N = 10000
_E = 320000
_D = 128
_DE = 16
_G = 250
_NPG = 40
_MLP = 256
_NC = 10
_BN_S = float(1.0 / np.sqrt(1.0 + 1e-5))
_DW = 8

_NW = 32
_PER_W = _E // _NW
_C1 = 112
_C2 = 128
_RPT = 624
_TAIL = _N - 16 * _RPT
_TAIL_OFF = 16 * _RPT

_mesh = plsc.VectorSubcoreMesh(core_axis_name="c", subcore_axis_name="s")


def _make_sc_agg(with_ea, C, deep=False):
    STEPS = _PER_W // C
    CT = _PER_W - STEPS * C
    LOOPN = (STEPS - 5) // 6
    outh_t = jax.ShapeDtypeStruct((2, _N, _D), jnp.float32)
    if with_ea:
        out_type = [outh_t,
                    jax.ShapeDtypeStruct((2, _N, _DE), jnp.float32),
                    jax.ShapeDtypeStruct((2, _N, _DW), jnp.float32)]
    else:
        out_type = outh_t
    scratch = [
        pltpu.VMEM((2, C), jnp.int32),
        pltpu.VMEM((2, C), jnp.int32),
        pltpu.VMEM((2, C), jnp.int32),
        pltpu.VMEM((C, _D), jnp.float32),
        pltpu.VMEM((C, _D), jnp.float32),
        pltpu.VMEM_SHARED((_N, _D), jnp.float32),
        pltpu.SemaphoreType.DMA,
        pltpu.SemaphoreType.DMA,
        pltpu.SemaphoreType.DMA,
        pltpu.SemaphoreType.DMA,
    ]
    if deep:
        scratch += [pltpu.VMEM((C, _D), jnp.float32),
                    pltpu.SemaphoreType.DMA]
    if with_ea:
        scratch += [
            pltpu.VMEM((C, _DE), jnp.float32),
            pltpu.VMEM((C, _DE), jnp.float32),
            pltpu.VMEM((C, _DW), jnp.float32),
            pltpu.VMEM_SHARED((_N, _DE), jnp.float32),
            pltpu.VMEM_SHARED((_N, _DW), jnp.float32),
            pltpu.SemaphoreType.DMA,
            pltpu.SemaphoreType.DMA,
            pltpu.SemaphoreType.DMA,
            pltpu.SemaphoreType.DMA,
            pltpu.SemaphoreType.DMA,
        ]
    scratch.append(pltpu.VMEM((2, CT), jnp.int32))

    @functools.partial(
        pl.kernel,
        mesh=_mesh,
        compiler_params=pltpu.CompilerParams(use_tc_tiling_on_sc=False),
        out_type=out_type,
        scratch_types=scratch,
    )
    def k(*refs):
        if with_ea:
            (h_hbm, idx_hbm, ea_hbm, ones_hbm, z128_hbm, z16_hbm, z1_hbm, idxt_hbm,
             outh_hbm, outea_hbm, outdeg_hbm,
             idx0, idx1, idx2, rows0, rows1, acch, isem, gsem, ss0, ss1,
             eab0, eab1, dones, accea, accdeg, easem, es0, es1, ds0, ds1, idxt) = refs
        else:
            if deep:
                (h_hbm, idx_hbm, z128_hbm, idxt_hbm,
                 outh_hbm,
                 idx0, idx1, idx2, rows0, rows1, acch, isem, gsem, ss0, ss1,
                 rows2, ss2, idxt) = refs
            else:
                (h_hbm, idx_hbm, z128_hbm, idxt_hbm,
                 outh_hbm,
                 idx0, idx1, idx2, rows0, rows1, acch, isem, gsem, ss0, ss1) = refs[:15]
                idxt = refs[15]
                rows2 = ss2 = None
            eab0 = eab1 = dones = accea = accdeg = easem = None
            es0 = es1 = ds0 = ds1 = None

        c = lax.axis_index("c")
        s = lax.axis_index("s")
        wid = s * 2 + c
        rbase = s * _RPT
        ebase = wid * _PER_W

        idxb = (idx0, idx1, idx2)
        rowsb = (rows0, rows1, rows2) if deep else (rows0, rows1)
        ssb = (ss0, ss1, ss2) if deep else (ss0, ss1)
        eabb = (eab0, eab1)
        esb = (es0, es1)
        dsb = (ds0, ds1)

        pltpu.sync_copy(z128_hbm.at[pl.ds(rbase, _RPT)], acch.at[pl.ds(rbase, _RPT)])
        if with_ea:
            pltpu.sync_copy(z16_hbm.at[pl.ds(rbase, _RPT)], accea.at[pl.ds(rbase, _RPT)])
            pltpu.sync_copy(z1_hbm.at[pl.ds(rbase, _RPT)], accdeg.at[pl.ds(rbase, _RPT)])
            pltpu.sync_copy(ones_hbm, dones)

        @pl.when(s == 0)
        def _():
            pltpu.sync_copy(z128_hbm.at[pl.ds(_TAIL_OFF, _TAIL)], acch.at[pl.ds(_TAIL_OFF, _TAIL)])
            if with_ea:
                pltpu.sync_copy(z16_hbm.at[pl.ds(_TAIL_OFF, _TAIL)], accea.at[pl.ds(_TAIL_OFF, _TAIL)])
                pltpu.sync_copy(z1_hbm.at[pl.ds(_TAIL_OFF, _TAIL)], accdeg.at[pl.ds(_TAIL_OFF, _TAIL)])

        plsc.subcore_barrier()

        def idx_load(i, b):
            pltpu.async_copy(idx_hbm.at[wid, i], idxb[b], isem)

        def idx_wait(i, b):
            pltpu.make_async_copy(idx_hbm.at[wid, i], idxb[b], isem).wait()

        def gather(ib, rb):
            pltpu.async_copy(h_hbm.at[idxb[ib].at[0]], rowsb[rb], gsem)

        def gather_wait(ib, rb):
            pltpu.make_async_copy(h_hbm.at[idxb[ib].at[0]], rowsb[rb], gsem).wait()

        def ea_load(i, b):
            pltpu.async_copy(ea_hbm.at[pl.ds(ebase + i * C, C)], eabb[b], easem)

        def ea_wait(i, b):
            pltpu.make_async_copy(ea_hbm.at[pl.ds(ebase + i * C, C)], eabb[b], easem).wait()

        def scats(ib, p):
            dst = idxb[ib].at[1]
            pltpu.async_copy(rowsb[p], acch.at[dst], ssb[p], add=True)
            if with_ea:
                pltpu.async_copy(eabb[p], accea.at[dst], esb[p], add=True)
                pltpu.async_copy(dones, accdeg.at[dst], dsb[p], add=True)

        def scats_drain(ib, p):
            dst = idxb[ib].at[1]
            pltpu.make_async_copy(rowsb[p], acch.at[dst], ssb[p]).wait()
            if with_ea:
                pltpu.make_async_copy(eabb[p], accea.at[dst], esb[p]).wait()
                pltpu.make_async_copy(dones, accdeg.at[dst], dsb[p]).wait()

        def phase(i, k_, j=None, last=False, skip_idx2=False):
            p = k_ % 2
            q = (k_ + 1) % 2
            ib = k_ % 3
            ibn = (k_ + 1) % 3
            ibp = (k_ + 2) % 3
            gather_wait(ib, p)
            if with_ea:
                ea_wait(i, p)
            scats(ib, p)

            def drain_prev():
                scats_drain(ibp, q)

            if j is None:
                drain_prev()
            else:
                pl.when(j >= 1)(drain_prev)

            if last:
                return
            if not skip_idx2:
                idx_load(i + 2, ibp)
            idx_wait(i + 1, ibn)
            gather(ibn, q)
            if with_ea:
                ea_load(i + 1, q)

        def phase_deep(i, k_, j=None, last=False, skip_idx2=False):
            r = k_ % 3
            rn = (k_ + 1) % 3
            rp = (k_ + 2) % 3
            gather_wait(r, r)
            if not last:
                idx_wait(i + 1, rn)
                gather(rn, rn)
            pltpu.async_copy(rowsb[r], acch.at[idxb[r].at[1]], ssb[r], add=True)

            def drain_prev():
                pltpu.make_async_copy(rowsb[rp], acch.at[idxb[rp].at[1]], ssb[rp]).wait()

            if j is None:
                drain_prev()
            else:
                pl.when(j >= 1)(drain_prev)
            if not (last or skip_idx2):
                idx_load(i + 2, rp)

        phase_fn = phase_deep if deep else phase

        idx_load(0, 0)
        idx_wait(0, 0)
        gather(0, 0)
        if with_ea:
            ea_load(0, 0)
        idx_load(1, 1)

        def body(j, carry):
            i6 = 6 * j
            phase_fn(i6 + 0, 0, j=j)
            phase_fn(i6 + 1, 1)
            phase_fn(i6 + 2, 2)
            phase_fn(i6 + 3, 3)
            phase_fn(i6 + 4, 4)
            phase_fn(i6 + 5, 5)
            return carry

        lax.fori_loop(0, LOOPN, body, 0)

        for i in range(6 * LOOPN, STEPS):
            phase_fn(i, i % 6, last=(i == STEPS - 1), skip_idx2=(i + 2 >= STEPS))
        if deep:
            scats_drain((STEPS - 1) % 3, (STEPS - 1) % 3)
        else:
            scats_drain((STEPS - 1) % 3, (STEPS - 1) % 2)

        if CT:
            tbase = ebase + STEPS * C
            pltpu.sync_copy(idxt_hbm.at[wid], idxt)
            pltpu.async_copy(h_hbm.at[idxt.at[0]], rows0.at[pl.ds(0, CT)], gsem)
            pltpu.make_async_copy(h_hbm.at[idxt.at[0]], rows0.at[pl.ds(0, CT)], gsem).wait()
            pltpu.async_copy(rows0.at[pl.ds(0, CT)], acch.at[idxt.at[1]], ss0, add=True)
            if with_ea:
                pltpu.sync_copy(ea_hbm.at[pl.ds(tbase, CT)], eab0.at[pl.ds(0, CT)])
                pltpu.async_copy(eab0.at[pl.ds(0, CT)], accea.at[idxt.at[1]], es0, add=True)
                pltpu.async_copy(dones.at[pl.ds(0, CT)], accdeg.at[idxt.at[1]], ds0, add=True)
            pltpu.make_async_copy(rows0.at[pl.ds(0, CT)], acch.at[idxt.at[1]], ss0).wait()
            if with_ea:
                pltpu.make_async_copy(eab0.at[pl.ds(0, CT)], accea.at[idxt.at[1]], es0).wait()
                pltpu.make_async_copy(dones.at[pl.ds(0, CT)], accdeg.at[idxt.at[1]], ds0).wait()

        plsc.subcore_barrier()
        pltpu.sync_copy(acch.at[pl.ds(rbase, _RPT)], outh_hbm.at[c, pl.ds(rbase, _RPT)])
        if with_ea:
            pltpu.sync_copy(accea.at[pl.ds(rbase, _RPT)], outea_hbm.at[c, pl.ds(rbase, _RPT)])
            pltpu.sync_copy(accdeg.at[pl.ds(rbase, _RPT)], outdeg_hbm.at[c, pl.ds(rbase, _RPT)])

        @pl.when(s == 0)
        def _():
            pltpu.sync_copy(acch.at[pl.ds(_TAIL_OFF, _TAIL)], outh_hbm.at[c, pl.ds(_TAIL_OFF, _TAIL)])
            if with_ea:
                pltpu.sync_copy(accea.at[pl.ds(_TAIL_OFF, _TAIL)], outea_hbm.at[c, pl.ds(_TAIL_OFF, _TAIL)])
                pltpu.sync_copy(accdeg.at[pl.ds(_TAIL_OFF, _TAIL)], outdeg_hbm.at[c, pl.ds(_TAIL_OFF, _TAIL)])

    return k


_sc_agg_first = _make_sc_agg(True, _C1)
_sc_agg = _make_sc_agg(False, _C2, deep=True)


_R = 1000


def _tc_layer(h, aggA, aggB, eaA, eaB, dgA, dgB, W, p3):
    def body(h_r, aA_r, aB_r, eA_r, eB_r, dA_r, dB_r, W_r, p_r, o_r):
        hb = h_r[...]
        agg = aA_r[...] + aB_r[...] + hb
        eav = eA_r[...] + eB_r[...] + 1.0
        deg = dA_r[..., 0:1] + dB_r[..., 0:1] + 1.0
        Wf = W_r[...]
        b = p_r[0:1, :]
        gs = p_r[1:2, :] * _BN_S
        be = p_r[2:3, :]
        out = (jnp.dot(hb, Wf[0:128], preferred_element_type=jnp.float32) + b) * deg
        out = out + jnp.dot(agg, Wf[128:256], preferred_element_type=jnp.float32)
        out = out + jnp.dot(eav, Wf[256:272], preferred_element_type=jnp.float32)
        hn = jnp.maximum(out, 0.0) * gs + be
        o_r[...] = jnp.maximum(hn, 0.0)

    return pl.pallas_call(
        body,
        grid=(_N // _R,),
        in_specs=[
            pl.BlockSpec((_R, _D), lambda i: (i, 0)),
            pl.BlockSpec((_R, _D), lambda i: (i, 0)),
            pl.BlockSpec((_R, _D), lambda i: (i, 0)),
            pl.BlockSpec((_R, _DE), lambda i: (i, 0)),
            pl.BlockSpec((_R, _DE), lambda i: (i, 0)),
            pl.BlockSpec((_R, _DW), lambda i: (i, 0)),
            pl.BlockSpec((_R, _DW), lambda i: (i, 0)),
            pl.BlockSpec((2 * _D + _DE, _D), lambda i: (0, 0)),
            pl.BlockSpec((3, _D), lambda i: (0, 0)),
        ],
        out_specs=pl.BlockSpec((_R, _D), lambda i: (i, 0)),
        out_shape=jax.ShapeDtypeStruct((_N, _D), jnp.float32),
    )(h, aggA, aggB, eaA, eaB, dgA, dgB, W, p3)


def _tc_layer2_head(h, aggA, aggB, eaA, eaB, dgA, dgB, W, p3,
                    wp, wm1, wm2, bias1, fc2_W, fc2_b):
    gpb = _R // _NPG

    def body(h_r, aA_r, aB_r, eA_r, eB_r, dA_r, dB_r, W_r, p_r,
             wp_r, w1_r, w2_r, b1_r, fw2_r, fb2_r, o_r,
             pool_s, e1_s, e2_s):
        i = pl.program_id(0)
        hb = h_r[...]
        agg = aA_r[...] + aB_r[...] + hb
        eav = eA_r[...] + eB_r[...] + 1.0
        deg = dA_r[..., 0:1] + dB_r[..., 0:1] + 1.0
        Wf = W_r[...]
        b = p_r[0:1, :]
        gs = p_r[1:2, :] * _BN_S
        be = p_r[2:3, :]
        out = (jnp.dot(hb, Wf[0:128], preferred_element_type=jnp.float32) + b) * deg
        out = out + jnp.dot(agg, Wf[128:256], preferred_element_type=jnp.float32)
        out = out + jnp.dot(eav, Wf[256:272], preferred_element_type=jnp.float32)
        hn = jnp.maximum(out, 0.0) * gs + be
        h2 = jnp.maximum(hn, 0.0)
        h2g = h2.reshape(gpb, _NPG, _D)
        pool_s[i] = jnp.sum(h2g, axis=1)
        e1_s[i] = hb.reshape(gpb, _NPG, _D)[:, 0, :]
        e2_s[i] = h2g[:, 0, :]

        @pl.when(i == _N // _R - 1)
        def _():
            pooled = pool_s[...].reshape(_G, _D)
            emb1 = e1_s[...].reshape(_G, _D)
            emb2 = e2_s[...].reshape(_G, _D)
            r = jnp.dot(pooled, wp_r[...], preferred_element_type=jnp.float32)
            r = r + jnp.dot(emb1, w1_r[...], preferred_element_type=jnp.float32)
            r = r + jnp.dot(emb2, w2_r[...], preferred_element_type=jnp.float32)
            r = jnp.maximum(r + b1_r[...], 0.0)
            o_r[...] = jnp.dot(r, fw2_r[...], preferred_element_type=jnp.float32) + fb2_r[...]

    blk = lambda i: (i, 0)
    zero2 = lambda i: (0, 0)
    return pl.pallas_call(
        body,
        grid=(_N // _R,),
        in_specs=[
            pl.BlockSpec((_R, _D), blk),
            pl.BlockSpec((_R, _D), blk),
            pl.BlockSpec((_R, _D), blk),
            pl.BlockSpec((_R, _DE), blk),
            pl.BlockSpec((_R, _DE), blk),
            pl.BlockSpec((_R, _DW), blk),
            pl.BlockSpec((_R, _DW), blk),
            pl.BlockSpec((2 * _D + _DE, _D), zero2),
            pl.BlockSpec((3, _D), zero2),
            pl.BlockSpec((_D, _MLP), zero2),
            pl.BlockSpec((_D, _MLP), zero2),
            pl.BlockSpec((_D, _MLP), zero2),
            pl.BlockSpec((1, _MLP), zero2),
            pl.BlockSpec((_MLP, _NC), zero2),
            pl.BlockSpec((1, _NC), zero2),
        ],
        out_specs=pl.BlockSpec((_G, _NC), zero2),
        out_shape=jax.ShapeDtypeStruct((_G, _NC), jnp.float32),
        scratch_shapes=[
            pltpu.VMEM((_N // _R, gpb, _D), jnp.float32),
            pltpu.VMEM((_N // _R, gpb, _D), jnp.float32),
            pltpu.VMEM((_N // _R, gpb, _D), jnp.float32),
        ],
    )(h, aggA, aggB, eaA, eaB, dgA, dgB, W, p3, wp, wm1, wm2, bias1, fc2_W, fc2_b)


def kernel(x, edge_attr, W1, b1, g1, be1, W2, b2, g2, be2,
           fc1_W, fc1_b, fc2_W, fc2_b, edge_index, batch):
    ei3 = edge_index.reshape(2, _NW, _PER_W)
    s1 = _PER_W // _C1
    s2 = _PER_W // _C2
    idx4a = ei3[:, :, :s1 * _C1].reshape(2, _NW, s1, _C1).transpose(1, 2, 0, 3)
    idxta = ei3[:, :, s1 * _C1:].transpose(1, 0, 2)
    idx4b = ei3[:, :, :s2 * _C2].reshape(2, _NW, s2, _C2).transpose(1, 2, 0, 3)
    idxtb = ei3[:, :, s2 * _C2:].transpose(1, 0, 2)
    ones_c = jnp.concatenate([jnp.ones((_C1, 1), jnp.float32), jnp.zeros((_C1, _DW - 1), jnp.float32)], axis=1)
    z128 = jnp.zeros((_N, _D), jnp.float32)
    z16 = jnp.zeros((_N, _DE), jnp.float32)
    z1 = jnp.zeros((_N, _DW), jnp.float32)

    aggh, aggea, aggdg = _sc_agg_first(x, idx4a, edge_attr, ones_c, z128, z16, z1, idxta)
    h1 = _tc_layer(x, aggh[0], aggh[1], aggea[0], aggea[1], aggdg[0], aggdg[1],
                   W1, jnp.stack([b1, g1, be1]))
    aggh2 = _sc_agg(h1, idx4b, z128, idxtb)

    wp = fc1_W[0:128]
    wm1 = fc1_W[129:257]
    wm2 = fc1_W[257:385]
    bias1 = (fc1_b + fc1_W[128]).reshape(1, _MLP)
    return _tc_layer2_head(h1, aggh2[0], aggh2[1], aggea[0], aggea[1],
                           aggdg[0], aggdg[1], W2, jnp.stack([b2, g2, be2]),
                           wp, wm1, wm2, bias1, fc2_W, fc2_b.reshape(1, _NC))


# speedup vs baseline: 12.7530x; 1.0058x over previous
import functools

import numpy as np
import jax
import jax.numpy as jnp
from jax import lax
from jax.experimental import pallas as pl
from jax.experimental.pallas import tpu as pltpu
from jax.experimental.pallas import tpu_sc as plsc

_N = 10000
_E = 320000
_D = 128
_DE = 16
_G = 250
_NPG = 40
_MLP = 256
_NC = 10
_BN_S = float(1.0 / np.sqrt(1.0 + 1e-5))
_DW = 8

_NW = 32
_PER_W = _E // _NW
_C1 = 112
_C2 = 128
_RPT = 624
_TAIL = _N - 16 * _RPT
_TAIL_OFF = 16 * _RPT

_mesh = plsc.VectorSubcoreMesh(core_axis_name="c", subcore_axis_name="s")


def _make_sc_agg(with_ea, C, deep=False):
    STEPS = _PER_W // C
    CT = _PER_W - STEPS * C
    LOOPN = (STEPS - 5) // 6
    outh_t = jax.ShapeDtypeStruct((2, _N, _D), jnp.float32)
    if with_ea:
        out_type = [outh_t,
                    jax.ShapeDtypeStruct((2, _N, _DE), jnp.float32),
                    jax.ShapeDtypeStruct((2, _N, _DW), jnp.float32)]
    else:
        out_type = outh_t
    scratch = [
        pltpu.VMEM((2, C), jnp.int32),
        pltpu.VMEM((2, C), jnp.int32),
        pltpu.VMEM((2, C), jnp.int32),
        pltpu.VMEM((C, _D), jnp.float32),
        pltpu.VMEM((C, _D), jnp.float32),
        pltpu.VMEM_SHARED((_N, _D), jnp.float32),
        pltpu.SemaphoreType.DMA,
        pltpu.SemaphoreType.DMA,
        pltpu.SemaphoreType.DMA,
        pltpu.SemaphoreType.DMA,
    ]
    if deep:
        scratch += [pltpu.VMEM((C, _D), jnp.float32),
                    pltpu.SemaphoreType.DMA]
    if with_ea:
        scratch += [
            pltpu.VMEM((C, _DE), jnp.float32),
            pltpu.VMEM((C, _DE), jnp.float32),
            pltpu.VMEM((C, _DW), jnp.float32),
            pltpu.VMEM_SHARED((_N, _DE), jnp.float32),
            pltpu.VMEM_SHARED((_N, _DW), jnp.float32),
            pltpu.SemaphoreType.DMA,
            pltpu.SemaphoreType.DMA,
            pltpu.SemaphoreType.DMA,
            pltpu.SemaphoreType.DMA,
            pltpu.SemaphoreType.DMA,
        ]
    scratch.append(pltpu.VMEM((2, CT), jnp.int32))

    @functools.partial(
        pl.kernel,
        mesh=_mesh,
        compiler_params=pltpu.CompilerParams(use_tc_tiling_on_sc=False),
        out_type=out_type,
        scratch_types=scratch,
    )
    def k(*refs):
        if with_ea:
            (h_hbm, idx_hbm, ea_hbm, ones_hbm, z128_hbm, z16_hbm, z1_hbm, idxt_hbm,
             outh_hbm, outea_hbm, outdeg_hbm,
             idx0, idx1, idx2, rows0, rows1, acch, isem, gsem, ss0, ss1,
             eab0, eab1, dones, accea, accdeg, easem, es0, es1, ds0, ds1, idxt) = refs
        else:
            if deep:
                (h_hbm, idx_hbm, z128_hbm, idxt_hbm,
                 outh_hbm,
                 idx0, idx1, idx2, rows0, rows1, acch, isem, gsem, ss0, ss1,
                 rows2, ss2, idxt) = refs
            else:
                (h_hbm, idx_hbm, z128_hbm, idxt_hbm,
                 outh_hbm,
                 idx0, idx1, idx2, rows0, rows1, acch, isem, gsem, ss0, ss1) = refs[:15]
                idxt = refs[15]
                rows2 = ss2 = None
            eab0 = eab1 = dones = accea = accdeg = easem = None
            es0 = es1 = ds0 = ds1 = None

        c = lax.axis_index("c")
        s = lax.axis_index("s")
        wid = s * 2 + c
        rbase = s * _RPT
        ebase = wid * _PER_W

        idxb = (idx0, idx1, idx2)
        rowsb = (rows0, rows1, rows2) if deep else (rows0, rows1)
        ssb = (ss0, ss1, ss2) if deep else (ss0, ss1)
        eabb = (eab0, eab1)
        esb = (es0, es1)
        dsb = (ds0, ds1)

        pltpu.sync_copy(z128_hbm.at[pl.ds(rbase, _RPT)], acch.at[pl.ds(rbase, _RPT)])
        if with_ea:
            pltpu.sync_copy(z16_hbm.at[pl.ds(rbase, _RPT)], accea.at[pl.ds(rbase, _RPT)])
            pltpu.sync_copy(z1_hbm.at[pl.ds(rbase, _RPT)], accdeg.at[pl.ds(rbase, _RPT)])
            pltpu.sync_copy(ones_hbm, dones)

        @pl.when(s == 0)
        def _():
            pltpu.sync_copy(z128_hbm.at[pl.ds(_TAIL_OFF, _TAIL)], acch.at[pl.ds(_TAIL_OFF, _TAIL)])
            if with_ea:
                pltpu.sync_copy(z16_hbm.at[pl.ds(_TAIL_OFF, _TAIL)], accea.at[pl.ds(_TAIL_OFF, _TAIL)])
                pltpu.sync_copy(z1_hbm.at[pl.ds(_TAIL_OFF, _TAIL)], accdeg.at[pl.ds(_TAIL_OFF, _TAIL)])

        plsc.subcore_barrier()

        def idx_load(i, b):
            pltpu.async_copy(idx_hbm.at[wid, i], idxb[b], isem)

        def idx_wait(i, b):
            pltpu.make_async_copy(idx_hbm.at[wid, i], idxb[b], isem).wait()

        def gather(ib, rb):
            pltpu.async_copy(h_hbm.at[idxb[ib].at[0]], rowsb[rb], gsem)

        def gather_wait(ib, rb):
            pltpu.make_async_copy(h_hbm.at[idxb[ib].at[0]], rowsb[rb], gsem).wait()

        def ea_load(i, b):
            pltpu.async_copy(ea_hbm.at[pl.ds(ebase + i * C, C)], eabb[b], easem)

        def ea_wait(i, b):
            pltpu.make_async_copy(ea_hbm.at[pl.ds(ebase + i * C, C)], eabb[b], easem).wait()

        def scats(ib, p):
            dst = idxb[ib].at[1]
            pltpu.async_copy(rowsb[p], acch.at[dst], ssb[p], add=True)
            if with_ea:
                pltpu.async_copy(eabb[p], accea.at[dst], esb[p], add=True)
                pltpu.async_copy(dones, accdeg.at[dst], dsb[p], add=True)

        def scats_drain(ib, p):
            dst = idxb[ib].at[1]
            pltpu.make_async_copy(rowsb[p], acch.at[dst], ssb[p]).wait()
            if with_ea:
                pltpu.make_async_copy(eabb[p], accea.at[dst], esb[p]).wait()
                pltpu.make_async_copy(dones, accdeg.at[dst], dsb[p]).wait()

        def phase(i, k_, j=None, last=False, skip_idx2=False):
            p = k_ % 2
            q = (k_ + 1) % 2
            ib = k_ % 3
            ibn = (k_ + 1) % 3
            ibp = (k_ + 2) % 3
            gather_wait(ib, p)
            if with_ea:
                ea_wait(i, p)
            scats(ib, p)

            def drain_prev():
                scats_drain(ibp, q)

            if j is None:
                drain_prev()
            else:
                pl.when(j >= 1)(drain_prev)

            if last:
                return
            if not skip_idx2:
                idx_load(i + 2, ibp)
            idx_wait(i + 1, ibn)
            gather(ibn, q)
            if with_ea:
                ea_load(i + 1, q)

        def phase_deep(i, k_, j=None, last=False, skip_idx2=False):
            r = k_ % 3
            rn = (k_ + 1) % 3
            rp = (k_ + 2) % 3
            gather_wait(r, r)
            if not last:
                idx_wait(i + 1, rn)
                gather(rn, rn)
            pltpu.async_copy(rowsb[r], acch.at[idxb[r].at[1]], ssb[r], add=True)

            def drain_prev():
                pltpu.make_async_copy(rowsb[rp], acch.at[idxb[rp].at[1]], ssb[rp]).wait()

            if j is None:
                drain_prev()
            else:
                pl.when(j >= 1)(drain_prev)
            if not (last or skip_idx2):
                idx_load(i + 2, rp)

        phase_fn = phase_deep if deep else phase

        idx_load(0, 0)
        idx_wait(0, 0)
        gather(0, 0)
        if with_ea:
            ea_load(0, 0)
        idx_load(1, 1)

        def body(j, carry):
            i6 = 6 * j
            phase_fn(i6 + 0, 0, j=j)
            phase_fn(i6 + 1, 1)
            phase_fn(i6 + 2, 2)
            phase_fn(i6 + 3, 3)
            phase_fn(i6 + 4, 4)
            phase_fn(i6 + 5, 5)
            return carry

        lax.fori_loop(0, LOOPN, body, 0)

        for i in range(6 * LOOPN, STEPS):
            phase_fn(i, i % 6, last=(i == STEPS - 1), skip_idx2=(i + 2 >= STEPS))
        if deep:
            scats_drain((STEPS - 1) % 3, (STEPS - 1) % 3)
        else:
            scats_drain((STEPS - 1) % 3, (STEPS - 1) % 2)

        if CT:
            tbase = ebase + STEPS * C
            pltpu.sync_copy(idxt_hbm.at[wid], idxt)
            pltpu.async_copy(h_hbm.at[idxt.at[0]], rows0.at[pl.ds(0, CT)], gsem)
            pltpu.make_async_copy(h_hbm.at[idxt.at[0]], rows0.at[pl.ds(0, CT)], gsem).wait()
            pltpu.async_copy(rows0.at[pl.ds(0, CT)], acch.at[idxt.at[1]], ss0, add=True)
            if with_ea:
                pltpu.sync_copy(ea_hbm.at[pl.ds(tbase, CT)], eab0.at[pl.ds(0, CT)])
                pltpu.async_copy(eab0.at[pl.ds(0, CT)], accea.at[idxt.at[1]], es0, add=True)
                pltpu.async_copy(dones.at[pl.ds(0, CT)], accdeg.at[idxt.at[1]], ds0, add=True)
            pltpu.make_async_copy(rows0.at[pl.ds(0, CT)], acch.at[idxt.at[1]], ss0).wait()
            if with_ea:
                pltpu.make_async_copy(eab0.at[pl.ds(0, CT)], accea.at[idxt.at[1]], es0).wait()
                pltpu.make_async_copy(dones.at[pl.ds(0, CT)], accdeg.at[idxt.at[1]], ds0).wait()

        plsc.subcore_barrier()
        pltpu.sync_copy(acch.at[pl.ds(rbase, _RPT)], outh_hbm.at[c, pl.ds(rbase, _RPT)])
        if with_ea:
            pltpu.sync_copy(accea.at[pl.ds(rbase, _RPT)], outea_hbm.at[c, pl.ds(rbase, _RPT)])
            pltpu.sync_copy(accdeg.at[pl.ds(rbase, _RPT)], outdeg_hbm.at[c, pl.ds(rbase, _RPT)])

        @pl.when(s == 0)
        def _():
            pltpu.sync_copy(acch.at[pl.ds(_TAIL_OFF, _TAIL)], outh_hbm.at[c, pl.ds(_TAIL_OFF, _TAIL)])
            if with_ea:
                pltpu.sync_copy(accea.at[pl.ds(_TAIL_OFF, _TAIL)], outea_hbm.at[c, pl.ds(_TAIL_OFF, _TAIL)])
                pltpu.sync_copy(accdeg.at[pl.ds(_TAIL_OFF, _TAIL)], outdeg_hbm.at[c, pl.ds(_TAIL_OFF, _TAIL)])

    return k


_sc_agg_first = _make_sc_agg(True, _C1)
_sc_agg = _make_sc_agg(False, _C2, deep=True)


_R = 2000


def _tc_layer(h, aggA, aggB, eaA, eaB, dgA, dgB, W, p3):
    def body(h_r, aA_r, aB_r, eA_r, eB_r, dA_r, dB_r, W_r, p_r, o_r):
        hb = h_r[...]
        agg = aA_r[...] + aB_r[...] + hb
        eav = eA_r[...] + eB_r[...] + 1.0
        deg = dA_r[..., 0:1] + dB_r[..., 0:1] + 1.0
        Wf = W_r[...]
        b = p_r[0:1, :]
        gs = p_r[1:2, :] * _BN_S
        be = p_r[2:3, :]
        out = (jnp.dot(hb, Wf[0:128], preferred_element_type=jnp.float32) + b) * deg
        out = out + jnp.dot(agg, Wf[128:256], preferred_element_type=jnp.float32)
        out = out + jnp.dot(eav, Wf[256:272], preferred_element_type=jnp.float32)
        hn = jnp.maximum(out, 0.0) * gs + be
        o_r[...] = jnp.maximum(hn, 0.0)

    return pl.pallas_call(
        body,
        grid=(_N // _R,),
        in_specs=[
            pl.BlockSpec((_R, _D), lambda i: (i, 0)),
            pl.BlockSpec((_R, _D), lambda i: (i, 0)),
            pl.BlockSpec((_R, _D), lambda i: (i, 0)),
            pl.BlockSpec((_R, _DE), lambda i: (i, 0)),
            pl.BlockSpec((_R, _DE), lambda i: (i, 0)),
            pl.BlockSpec((_R, _DW), lambda i: (i, 0)),
            pl.BlockSpec((_R, _DW), lambda i: (i, 0)),
            pl.BlockSpec((2 * _D + _DE, _D), lambda i: (0, 0)),
            pl.BlockSpec((3, _D), lambda i: (0, 0)),
        ],
        out_specs=pl.BlockSpec((_R, _D), lambda i: (i, 0)),
        out_shape=jax.ShapeDtypeStruct((_N, _D), jnp.float32),
    )(h, aggA, aggB, eaA, eaB, dgA, dgB, W, p3)


def _tc_layer2_head(h, aggA, aggB, eaA, eaB, dgA, dgB, W, p3,
                    wp, wm1, wm2, bias1, fc2_W, fc2_b):
    gpb = _R // _NPG

    def body(h_r, aA_r, aB_r, eA_r, eB_r, dA_r, dB_r, W_r, p_r,
             wp_r, w1_r, w2_r, b1_r, fw2_r, fb2_r, o_r,
             pool_s, e1_s, e2_s):
        i = pl.program_id(0)
        hb = h_r[...]
        agg = aA_r[...] + aB_r[...] + hb
        eav = eA_r[...] + eB_r[...] + 1.0
        deg = dA_r[..., 0:1] + dB_r[..., 0:1] + 1.0
        Wf = W_r[...]
        b = p_r[0:1, :]
        gs = p_r[1:2, :] * _BN_S
        be = p_r[2:3, :]
        out = (jnp.dot(hb, Wf[0:128], preferred_element_type=jnp.float32) + b) * deg
        out = out + jnp.dot(agg, Wf[128:256], preferred_element_type=jnp.float32)
        out = out + jnp.dot(eav, Wf[256:272], preferred_element_type=jnp.float32)
        hn = jnp.maximum(out, 0.0) * gs + be
        h2 = jnp.maximum(hn, 0.0)
        h2g = h2.reshape(gpb, _NPG, _D)
        pool_s[i] = jnp.sum(h2g, axis=1)
        e1_s[i] = hb.reshape(gpb, _NPG, _D)[:, 0, :]
        e2_s[i] = h2g[:, 0, :]

        @pl.when(i == _N // _R - 1)
        def _():
            pooled = pool_s[...].reshape(_G, _D)
            emb1 = e1_s[...].reshape(_G, _D)
            emb2 = e2_s[...].reshape(_G, _D)
            r = jnp.dot(pooled, wp_r[...], preferred_element_type=jnp.float32)
            r = r + jnp.dot(emb1, w1_r[...], preferred_element_type=jnp.float32)
            r = r + jnp.dot(emb2, w2_r[...], preferred_element_type=jnp.float32)
            r = jnp.maximum(r + b1_r[...], 0.0)
            o_r[...] = jnp.dot(r, fw2_r[...], preferred_element_type=jnp.float32) + fb2_r[...]

    blk = lambda i: (i, 0)
    zero2 = lambda i: (0, 0)
    return pl.pallas_call(
        body,
        grid=(_N // _R,),
        in_specs=[
            pl.BlockSpec((_R, _D), blk),
            pl.BlockSpec((_R, _D), blk),
            pl.BlockSpec((_R, _D), blk),
            pl.BlockSpec((_R, _DE), blk),
            pl.BlockSpec((_R, _DE), blk),
            pl.BlockSpec((_R, _DW), blk),
            pl.BlockSpec((_R, _DW), blk),
            pl.BlockSpec((2 * _D + _DE, _D), zero2),
            pl.BlockSpec((3, _D), zero2),
            pl.BlockSpec((_D, _MLP), zero2),
            pl.BlockSpec((_D, _MLP), zero2),
            pl.BlockSpec((_D, _MLP), zero2),
            pl.BlockSpec((1, _MLP), zero2),
            pl.BlockSpec((_MLP, _NC), zero2),
            pl.BlockSpec((1, _NC), zero2),
        ],
        out_specs=pl.BlockSpec((_G, _NC), zero2),
        out_shape=jax.ShapeDtypeStruct((_G, _NC), jnp.float32),
        scratch_shapes=[
            pltpu.VMEM((_N // _R, gpb, _D), jnp.float32),
            pltpu.VMEM((_N // _R, gpb, _D), jnp.float32),
            pltpu.VMEM((_N // _R, gpb, _D), jnp.float32),
        ],
    )(h, aggA, aggB, eaA, eaB, dgA, dgB, W, p3, wp, wm1, wm2, bias1, fc2_W, fc2_b)


def kernel(x, edge_attr, W1, b1, g1, be1, W2, b2, g2, be2,
           fc1_W, fc1_b, fc2_W, fc2_b, edge_index, batch):
    ei3 = edge_index.reshape(2, _NW, _PER_W)
    s1 = _PER_W // _C1
    s2 = _PER_W // _C2
    idx4a = ei3[:, :, :s1 * _C1].reshape(2, _NW, s1, _C1).transpose(1, 2, 0, 3)
    idxta = ei3[:, :, s1 * _C1:].transpose(1, 0, 2)
    idx4b = ei3[:, :, :s2 * _C2].reshape(2, _NW, s2, _C2).transpose(1, 2, 0, 3)
    idxtb = ei3[:, :, s2 * _C2:].transpose(1, 0, 2)
    ones_c = jnp.concatenate([jnp.ones((_C1, 1), jnp.float32), jnp.zeros((_C1, _DW - 1), jnp.float32)], axis=1)
    z128 = jnp.zeros((_N, _D), jnp.float32)
    z16 = jnp.zeros((_N, _DE), jnp.float32)
    z1 = jnp.zeros((_N, _DW), jnp.float32)

    aggh, aggea, aggdg = _sc_agg_first(x, idx4a, edge_attr, ones_c, z128, z16, z1, idxta)
    h1 = _tc_layer(x, aggh[0], aggh[1], aggea[0], aggea[1], aggdg[0], aggdg[1],
                   W1, jnp.stack([b1, g1, be1]))
    aggh2 = _sc_agg(h1, idx4b, z128, idxtb)

    wp = fc1_W[0:128]
    wm1 = fc1_W[129:257]
    wm2 = fc1_W[257:385]
    bias1 = (fc1_b + fc1_W[128]).reshape(1, _MLP)
    return _tc_layer2_head(h1, aggh2[0], aggh2[1], aggea[0], aggea[1],
                           aggdg[0], aggdg[1], W2, jnp.stack([b2, g2, be2]),
                           wp, wm1, wm2, bias1, fc2_W, fc2_b.reshape(1, _NC))
